# baseline (speedup 1.0000x reference)
import functools

import jax
import jax.numpy as jnp
import numpy as np
from jax.experimental import pallas as pl
from jax.experimental.pallas import tpu as pltpu
from jax.experimental.shard_map import shard_map
from jax.sharding import Mesh, PartitionSpec as P

BSZ = 8
WIN = 100
NN = 512
IN_CH = 64
HID = 16
HEADS = 4
OUT_CH = 64

_CN = (((1,), (1,)), ((), ()))


def _dot_t(a, b):
    return jax.lax.dot_general(a, b, _CN, preferred_element_type=jnp.float32)


def _attend(x_own, xl0, cnt, wr, br, att_ref, att_v, heads, ch):
    xr = _dot_t(x_own, wr) + br
    xlT = xl0.T
    outs = []
    for h in range(heads):
        sl = slice(h * ch, (h + 1) * ch)
        ar = _dot_t(xr[:, sl], att_v[:, sl]) * 0.6
        al = _dot_t(att_v[:, sl], xl0[:, sl]) * 0.6
        s = ar + al
        for c in range(ch):
            k = h * ch + c
            z = xr[:, k:k + 1] + xlT[k:k + 1, :]
            s = s + (0.4 * att_ref[h, c]) * jnp.abs(z)
        sm = jnp.where(cnt > 0, s, -jnp.inf)
        amax = jnp.max(sm, axis=1, keepdims=True)
        p = cnt * jnp.exp(sm - amax)
        den = jnp.sum(p, axis=1, keepdims=True) + 1e-16
        o = jnp.dot(p, xl0[:, sl], preferred_element_type=jnp.float32) / den
        outs.append(o)
    return outs[0] if heads == 1 else jnp.concatenate(outs, axis=1)


def _stage_a(att1_ref, xt_ref, xt0_ref, cnt_ref, wemb_ref, bemb_ref,
             wl1_ref, bl1_ref, wr1_ref, br1_ref, att1v_ref,
             att_out_ref, xl1_tail_ref):
    x = _dot_t(xt_ref[...], wemb_ref[...]) + bemb_ref[...]
    xl1 = _dot_t(x, wl1_ref[...]) + bl1_ref[...]
    x_own = _dot_t(xt0_ref[...], wemb_ref[...]) + bemb_ref[...]
    att_out_ref[...] = _attend(x_own, xl1[:NN], cnt_ref[...],
                               wr1_ref[...], br1_ref[...],
                               att1_ref, att1v_ref[...], HEADS, HID)
    xl1_tail_ref[...] = xl1[NN:]


def _stage_b(att2_ref, h1_ref, own1_ref, cnt_ref, bias1_ref,
             wl2_ref, bl2_ref, wr2_ref, br2_ref, att2v_ref,
             part0_ref, lin_ref):
    def elu(v):
        return jnp.where(v > 0, v, jnp.exp(v) - 1.0)
    x2 = elu(h1_ref[...] + bias1_ref[...])
    xl2 = _dot_t(x2, wl2_ref[...]) + bl2_ref[...]
    x2_own = elu(own1_ref[...] + bias1_ref[...])
    o2 = _attend(x2_own, xl2[:NN], cnt_ref[...],
                 wr2_ref[...], br2_ref[...],
                 att2_ref, att2v_ref[...], 1, OUT_CH)
    part0_ref[...] = jnp.sum(o2, axis=0, keepdims=True)
    lin_ref[...] = jnp.mean(
        xl2[NN:].reshape(BSZ - 1, NN, OUT_CH), axis=1)


def kernel(input, adj_mtx, W_emb, b_emb, Wl1, bl1, Wr1, br1, att1, bias1,
           Wl2, bl2, Wr2, br2, att2, bias2):
    xt = jnp.swapaxes(input, 1, 2).reshape(BSZ * NN, WIN)
    cnt = (BSZ * (adj_mtx != 0).astype(jnp.float32).T
           + jnp.eye(NN, dtype=jnp.float32))
    xt0 = xt[:NN]

    devs = jax.devices()
    ndev = 2 if len(devs) >= 2 else 1
    jb = NN // ndev
    mesh = Mesh(np.array(devs[:ndev]), ("d",))

    smem = pl.BlockSpec(memory_space=pltpu.SMEM)
    row = lambda v: v.reshape(1, -1)
    f32 = jnp.float32

    def stage_a_call(xt_, xt0_d, cnt_d, wemb, bemb, wl1_, bl1_, wr1_, br1_,
                     a1, a1v):
        return pl.pallas_call(
            _stage_a,
            in_specs=[smem] + [pl.BlockSpec()] * 10,
            out_specs=(pl.BlockSpec(), pl.BlockSpec()),
            out_shape=(jax.ShapeDtypeStruct((jb, HEADS * HID), f32),
                       jax.ShapeDtypeStruct(((BSZ - 1) * NN, HEADS * HID), f32)),
        )(a1, xt_, xt0_d, cnt_d, wemb, bemb, wl1_, bl1_, wr1_, br1_, a1v)

    def stage_b_call(h1, own1, cnt_d, b1, wl2_, bl2_, wr2_, br2_, a2, a2v):
        return pl.pallas_call(
            _stage_b,
            in_specs=[smem] + [pl.BlockSpec()] * 9,
            out_specs=(pl.BlockSpec(), pl.BlockSpec()),
            out_shape=(jax.ShapeDtypeStruct((1, OUT_CH), f32),
                       jax.ShapeDtypeStruct((BSZ - 1, OUT_CH), f32)),
        )(a2, h1, own1, cnt_d, b1, wl2_, bl2_, wr2_, br2_, a2v)

    rep = P(None, None)
    shd = P("d", None)

    @functools.partial(
        shard_map, mesh=mesh,
        in_specs=(rep, shd, shd) + (rep,) * 15,
        out_specs=P(None, None), check_rep=False)
    def run(xt_, xt0_d, cnt_d, wemb, bemb, wl1_, bl1_, wr1_, br1_, a1, a1v,
            b1, wl2_, bl2_, wr2_, br2_, a2, a2v):
        att1_d, xl1_tail = stage_a_call(
            xt_, xt0_d, cnt_d, wemb, bemb, wl1_, bl1_, wr1_, br1_, a1, a1v)
        att1_full = jax.lax.all_gather(att1_d, "d", axis=0, tiled=True)
        h1 = jnp.concatenate([att1_full, xl1_tail], axis=0)
        part0, lin = stage_b_call(
            h1, att1_d, cnt_d, b1, wl2_, bl2_, wr2_, br2_, a2, a2v)
        out0 = jax.lax.psum(part0, "d") / NN
        return jnp.concatenate([out0, lin], axis=0)

    out = run(xt, xt0, cnt, W_emb, row(b_emb), Wl1, row(bl1), Wr1, row(br1),
              att1, att1.reshape(1, HEADS * HID), row(bias1),
              Wl2, row(bl2), Wr2, row(br2),
              att2, att2.reshape(1, OUT_CH))
    return out + bias2.reshape(1, OUT_CH)


# speedup vs baseline: 8.8911x; 8.8911x over previous
import jax
import jax.numpy as jnp
from jax.experimental import pallas as pl
from jax.experimental.pallas import tpu as pltpu

BSZ = 8
WIN = 100
NN = 512
IN_CH = 64
HID = 16
HEADS = 4
OUT_CH = 64

_CN = (((1,), (1,)), ((), ()))


def _dot_t(a, b):
    return jax.lax.dot_general(a, b, _CN, preferred_element_type=jnp.float32)


def _attend(x0, xl0, cnt, mask, wr, br, att_ref, att_v, heads, ch):
    xr = _dot_t(x0, wr) + br
    xrT = xr.T
    outs = []
    for h in range(heads):
        sl = slice(h * ch, (h + 1) * ch)
        al = _dot_t(xl0[:, sl], att_v[:, sl]) * 0.6
        ar = _dot_t(att_v[:, sl], xr[:, sl]) * 0.6
        s = al + ar
        for c in range(ch):
            k = h * ch + c
            z = xl0[:, k:k + 1] + xrT[k:k + 1, :]
            s = s + (0.4 * att_ref[h, c]) * jnp.abs(z)
        sm = jnp.where(mask, s, -jnp.inf)
        amax = jnp.max(sm, axis=0, keepdims=True)
        p = cnt * jnp.exp(sm - amax)
        p = p * (1.0 / (jnp.sum(p, axis=0, keepdims=True) + 1e-16))
        o = jax.lax.dot_general(p, xl0[:, sl], (((0,), (0,)), ((), ())),
                                preferred_element_type=jnp.float32)
        outs.append(o)
    return outs[0] if heads == 1 else jnp.concatenate(outs, axis=1)


def _enc_kernel(att1_ref, att2_ref, inp_ref, adj_ref, wemb_ref, bemb_ref,
                wl1_ref, bl1_ref, wr1_ref, br1_ref, bias1_ref,
                wl2_ref, bl2_ref, wr2_ref, br2_ref, bias2_ref,
                att1v_ref, att2v_ref, out_ref):
    edge = adj_ref[...] != 0
    diag = (jax.lax.broadcasted_iota(jnp.int32, (NN, NN), 0)
            == jax.lax.broadcasted_iota(jnp.int32, (NN, NN), 1))
    cnt = jnp.where(edge, float(BSZ), 0.0) + jnp.where(diag, 1.0, 0.0)
    mask = jnp.logical_or(edge, diag)
    xs = [jax.lax.dot_general(inp_ref[b], wemb_ref[...],
                              (((0,), (1,)), ((), ())),
                              preferred_element_type=jnp.float32)
          for b in range(BSZ)]
    x = jnp.concatenate(xs, axis=0) + bemb_ref[...]

    xl1 = _dot_t(x, wl1_ref[...]) + bl1_ref[...]
    att_out1 = _attend(x[:NN], xl1[:NN], cnt, mask, wr1_ref[...],
                       br1_ref[...], att1_ref, att1v_ref[...], HEADS, HID)
    h1 = jnp.concatenate([att_out1, xl1[NN:]], axis=0) + bias1_ref[...]
    x2 = jnp.where(h1 > 0, h1, jnp.exp(h1) - 1.0)

    xl2 = _dot_t(x2, wl2_ref[...]) + bl2_ref[...]
    att_out2 = _attend(x2[:NN], xl2[:NN], cnt, mask, wr2_ref[...],
                       br2_ref[...], att2_ref, att2v_ref[...], 1, OUT_CH)
    h2 = jnp.concatenate([att_out2, xl2[NN:]], axis=0) + bias2_ref[...]

    out_ref[...] = jnp.mean(h2.reshape(BSZ, NN, OUT_CH), axis=1)


def kernel(input, adj_mtx, W_emb, b_emb, Wl1, bl1, Wr1, br1, att1, bias1,
           Wl2, bl2, Wr2, br2, att2, bias2):
    smem = pl.BlockSpec(memory_space=pltpu.SMEM)
    row = lambda v: v.reshape(1, -1)

    return pl.pallas_call(
        _enc_kernel,
        in_specs=[smem, smem] + [pl.BlockSpec()] * 16,
        out_specs=pl.BlockSpec(),
        out_shape=jax.ShapeDtypeStruct((BSZ, OUT_CH), jnp.float32),
    )(att1, att2, input, adj_mtx, W_emb, row(b_emb),
      Wl1, row(bl1), Wr1, row(br1), row(bias1),
      Wl2, row(bl2), Wr2, row(br2), row(bias2),
      att1.reshape(1, HEADS * HID), att2.reshape(1, OUT_CH))


# speedup vs baseline: 10.0413x; 1.1294x over previous
import jax
import jax.numpy as jnp
from jax.experimental import pallas as pl
from jax.experimental.pallas import tpu as pltpu

BSZ = 8
WIN = 100
NN = 512
IN_CH = 64
HID = 16
HEADS = 4
OUT_CH = 64

_CN = (((1,), (1,)), ((), ()))


def _dot_t(a, b):
    return jax.lax.dot_general(a, b, _CN, preferred_element_type=jnp.float32)


def _attend(x0, xl0, cnt, wr, br, att_ref, att_v, heads, ch):
    xr = _dot_t(x0, wr) + br
    xlT = xl0.T
    outs = []
    for h in range(heads):
        sl = slice(h * ch, (h + 1) * ch)
        ar = _dot_t(xr[:, sl], att_v[:, sl]) * 0.6
        al = _dot_t(att_v[:, sl], xl0[:, sl]) * 0.6
        s = ar + al
        for c in range(ch):
            k = h * ch + c
            z = xr[:, k:k + 1] + xlT[k:k + 1, :]
            s = s + (0.4 * att_ref[h, c]) * jnp.abs(z)
        sm = jnp.where(cnt > 0, s, -jnp.inf)
        amax = jnp.max(sm, axis=1, keepdims=True)
        p = cnt * jnp.exp(sm - amax)
        den = jnp.sum(p, axis=1, keepdims=True) + 1e-16
        o = jnp.dot(p, xl0[:, h * ch:(h + 1) * ch],
                    preferred_element_type=jnp.float32) / den
        outs.append(o)
    return outs[0] if heads == 1 else jnp.concatenate(outs, axis=1)


def _enc_kernel(att1_ref, att2_ref, xt_ref, cnt_ref, wemb_ref, bemb_ref,
                wl1_ref, bl1_ref, wr1_ref, br1_ref, bias1_ref,
                wl2_ref, bl2_ref, wr2_ref, br2_ref, bias2_ref,
                att1v_ref, att2v_ref, out_ref):
    cnt = cnt_ref[...]
    xs = [jax.lax.dot_general(xt_ref[b], wemb_ref[...],
                              (((0,), (1,)), ((), ())),
                              preferred_element_type=jnp.float32)
          for b in range(BSZ)]
    x = jnp.concatenate(xs, axis=0) + bemb_ref[...]

    xl1 = _dot_t(x, wl1_ref[...]) + bl1_ref[...]
    att_out1 = _attend(x[:NN], xl1[:NN], cnt, wr1_ref[...], br1_ref[...],
                       att1_ref, att1v_ref[...], HEADS, HID)
    h1 = jnp.concatenate([att_out1, xl1[NN:]], axis=0) + bias1_ref[...]
    x2 = jnp.where(h1 > 0, h1, jnp.exp(h1) - 1.0)

    xl2 = _dot_t(x2, wl2_ref[...]) + bl2_ref[...]
    att_out2 = _attend(x2[:NN], xl2[:NN], cnt, wr2_ref[...], br2_ref[...],
                       att2_ref, att2v_ref[...], 1, OUT_CH)
    h2 = jnp.concatenate([att_out2, xl2[NN:]], axis=0) + bias2_ref[...]

    out_ref[...] = jnp.mean(h2.reshape(BSZ, NN, OUT_CH), axis=1)


def kernel(input, adj_mtx, W_emb, b_emb, Wl1, bl1, Wr1, br1, att1, bias1,
           Wl2, bl2, Wr2, br2, att2, bias2):
    cnt = (BSZ * (adj_mtx != 0).astype(jnp.float32).T
           + jnp.eye(NN, dtype=jnp.float32))

    smem = pl.BlockSpec(memory_space=pltpu.SMEM)
    row = lambda v: v.reshape(1, -1)

    return pl.pallas_call(
        _enc_kernel,
        in_specs=[smem, smem] + [pl.BlockSpec()] * 16,
        out_specs=pl.BlockSpec(),
        out_shape=jax.ShapeDtypeStruct((BSZ, OUT_CH), jnp.float32),
    )(att1, att2, input, cnt, W_emb, row(b_emb),
      Wl1, row(bl1), Wr1, row(br1), row(bias1),
      Wl2, row(bl2), Wr2, row(br2), row(bias2),
      att1.reshape(1, HEADS * HID), att2.reshape(1, OUT_CH))


# speedup vs baseline: 10.1089x; 1.0067x over previous
import jax
import jax.numpy as jnp
from jax.experimental import pallas as pl
from jax.experimental.pallas import tpu as pltpu

BSZ = 8
WIN = 100
NN = 512
IN_CH = 64
HID = 16
HEADS = 4
OUT_CH = 64

_CN = (((1,), (1,)), ((), ()))


def _dot_t(a, b):
    return jax.lax.dot_general(a, b, _CN, preferred_element_type=jnp.float32)


def _attend(x0, xl0, cnt, wr, br, att_ref, att_v, heads, ch):
    xr = _dot_t(x0, wr) + br
    xlT = xl0.T
    outs = []
    for h in range(heads):
        sl = slice(h * ch, (h + 1) * ch)
        ar = _dot_t(xr[:, sl], att_v[:, sl]) * 0.6
        al = _dot_t(att_v[:, sl], xl0[:, sl]) * 0.6
        s = ar + al
        for c in range(ch):
            k = h * ch + c
            z = xr[:, k:k + 1] + xlT[k:k + 1, :]
            s = s + (0.4 * att_ref[h, c]) * jnp.abs(z)
        amax = jnp.max(s, axis=1, keepdims=True)
        p = cnt * jnp.exp(s - amax)
        den = jnp.sum(p, axis=1, keepdims=True) + 1e-16
        o = jnp.dot(p, xl0[:, h * ch:(h + 1) * ch],
                    preferred_element_type=jnp.float32) / den
        outs.append(o)
    return outs[0] if heads == 1 else jnp.concatenate(outs, axis=1)


def _enc_kernel(att1_ref, att2_ref, xt_ref, cnt_ref, wemb_ref, bemb_ref,
                wl1_ref, bl1_ref, wr1_ref, br1_ref, bias1_ref,
                wl2_ref, bl2_ref, wr2_ref, br2_ref, bias2_ref,
                att1v_ref, att2v_ref, out_ref):
    cnt = cnt_ref[...]
    xs = [jax.lax.dot_general(xt_ref[b], wemb_ref[...],
                              (((0,), (1,)), ((), ())),
                              preferred_element_type=jnp.float32)
          for b in range(BSZ)]
    x = jnp.concatenate(xs, axis=0) + bemb_ref[...]

    xl1 = _dot_t(x, wl1_ref[...]) + bl1_ref[...]
    att_out1 = _attend(x[:NN], xl1[:NN], cnt, wr1_ref[...], br1_ref[...],
                       att1_ref, att1v_ref[...], HEADS, HID)
    h1 = jnp.concatenate([att_out1, xl1[NN:]], axis=0) + bias1_ref[...]
    x2 = jnp.where(h1 > 0, h1, jnp.exp(h1) - 1.0)

    xl2 = _dot_t(x2, wl2_ref[...]) + bl2_ref[...]
    att_out2 = _attend(x2[:NN], xl2[:NN], cnt, wr2_ref[...], br2_ref[...],
                       att2_ref, att2v_ref[...], 1, OUT_CH)
    h2 = jnp.concatenate([att_out2, xl2[NN:]], axis=0) + bias2_ref[...]

    out_ref[...] = jnp.mean(h2.reshape(BSZ, NN, OUT_CH), axis=1)


def kernel(input, adj_mtx, W_emb, b_emb, Wl1, bl1, Wr1, br1, att1, bias1,
           Wl2, bl2, Wr2, br2, att2, bias2):
    cnt = (BSZ * (adj_mtx != 0).astype(jnp.float32).T
           + jnp.eye(NN, dtype=jnp.float32))

    smem = pl.BlockSpec(memory_space=pltpu.SMEM)
    row = lambda v: v.reshape(1, -1)

    return pl.pallas_call(
        _enc_kernel,
        in_specs=[smem, smem] + [pl.BlockSpec()] * 16,
        out_specs=pl.BlockSpec(),
        out_shape=jax.ShapeDtypeStruct((BSZ, OUT_CH), jnp.float32),
    )(att1, att2, input, cnt, W_emb, row(b_emb),
      Wl1, row(bl1), Wr1, row(br1), row(bias1),
      Wl2, row(bl2), Wr2, row(br2), row(bias2),
      att1.reshape(1, HEADS * HID), att2.reshape(1, OUT_CH))
